# baseline (speedup 1.0000x reference)
import jax
import jax.numpy as jnp
from jax.experimental import pallas as pl
from jax.experimental.pallas import tpu as pltpu

_BM = 1024


def _fused(x_ref, w1_ref, b1_ref, w2_ref, b2_ref, w3_ref, b3_ref,
           dw1_ref, db1_ref, dw2_ref, db2_ref, dw3_ref, db3_ref, cb_ref,
           pred_ref, counts_ref, l1acc_ref, sqacc_ref,
           cbh_ref, cbm_ref, cbl_ref_in):
    i = pl.program_id(0)
    bm = x_ref.shape[0]
    G, K, D = cb_ref.shape

    @pl.when(i == 0)
    def _init():
        counts_ref[...] = jnp.zeros_like(counts_ref)
        l1acc_ref[...] = jnp.zeros_like(l1acc_ref)
        sqacc_ref[...] = jnp.zeros_like(sqacc_ref)
        cb0 = cb_ref[...]
        hi = cb0.astype(jnp.bfloat16)
        r1 = cb0 - hi.astype(jnp.float32)
        mid = r1.astype(jnp.bfloat16)
        lo = r1 - mid.astype(jnp.float32)
        cbh_ref[...] = hi
        cbm_ref[...] = mid
        cbl_ref_in[...] = lo.astype(jnp.bfloat16)

    x = x_ref[...]
    h = jnp.maximum(
        jnp.dot(x, w1_ref[...], preferred_element_type=jnp.float32) + b1_ref[...], 0.0)
    h = jnp.maximum(
        jnp.dot(h, w2_ref[...], preferred_element_type=jnp.float32) + b2_ref[...], 0.0)
    z = jnp.dot(h, w3_ref[...], preferred_element_type=jnp.float32) + b3_ref[...]

    residual = z
    quantized = jnp.zeros_like(z)
    iota = jax.lax.broadcasted_iota(jnp.int32, (bm, K), 1)
    for g in range(G):
        cb = cb_ref[g]
        rsq = jnp.sum(residual * residual, axis=1, keepdims=True)
        csq = jnp.sum(cb * cb, axis=1)[None, :]
        mm = jax.lax.dot_general(residual, cb, (((1,), (1,)), ((), ())),
                                 preferred_element_type=jnp.float32)
        dist = rsq - 2.0 * mm + csq
        m = jnp.min(dist, axis=1, keepdims=True)
        idxv = jnp.min(jnp.where(dist == m, iota, K), axis=1, keepdims=True)
        onehot = (iota == idxv).astype(jnp.bfloat16)
        if g < G - 1:
            q = (jnp.dot(onehot, cbh_ref[g], preferred_element_type=jnp.float32)
                 + jnp.dot(onehot, cbm_ref[g], preferred_element_type=jnp.float32)
                 + jnp.dot(onehot, cbl_ref_in[g], preferred_element_type=jnp.float32))
        else:
            q = (jnp.dot(onehot, cbh_ref[g], preferred_element_type=jnp.float32)
                 + jnp.dot(onehot, cbm_ref[g], preferred_element_type=jnp.float32))
        quantized = quantized + q
        residual = residual - q
        ones_row = jnp.ones((1, bm), jnp.bfloat16)
        counts_ref[g:g + 1, :] = counts_ref[g:g + 1, :] + jnp.dot(
            ones_row, onehot, preferred_element_type=jnp.float32)

    z_q = z + (quantized - z)
    hd = jnp.maximum(
        jnp.dot(z_q, dw1_ref[...], preferred_element_type=jnp.float32) + db1_ref[...], 0.0)
    hd = jnp.maximum(
        jnp.dot(hd, dw2_ref[...], preferred_element_type=jnp.float32) + db2_ref[...], 0.0)
    pred = jnp.dot(hd, dw3_ref[...], preferred_element_type=jnp.float32) + db3_ref[...]
    pred_ref[...] = pred

    diff = z - quantized
    sqacc_ref[...] = sqacc_ref[...] + jnp.sum(diff * diff).reshape(1, 1)
    l1acc_ref[...] = l1acc_ref[...] + jnp.sum(jnp.abs(pred - x)).reshape(1, 1)


def kernel(actions, enc_W1, enc_b1, enc_W2, enc_b2, enc_W3, enc_b3,
           dec_W1, dec_b1, dec_W2, dec_b2, dec_W3, dec_b3, codebooks):
    Bsz, T, A = actions.shape
    Din = T * A
    H = enc_W1.shape[1]
    G, K, D = codebooks.shape
    bm = _BM
    Bh = Bsz // 2
    grid = Bh // bm

    b1 = enc_b1.reshape(1, H)
    b2 = enc_b2.reshape(1, H)
    b3 = enc_b3.reshape(1, D)
    db1 = dec_b1.reshape(1, H)
    db2 = dec_b2.reshape(1, H)
    db3 = dec_b3.reshape(1, Din)

    full = lambda shp: pl.BlockSpec(shp, lambda i: tuple(0 for _ in shp))
    scalar_spec = pl.BlockSpec((1, 1), lambda i: (0, 0))

    def _half(xh):
        return pl.pallas_call(
            _fused,
            grid=(grid,),
            in_specs=[
                pl.BlockSpec((bm, Din), lambda i: (i, 0)),
                full(enc_W1.shape), full(b1.shape),
                full(enc_W2.shape), full(b2.shape),
                full(enc_W3.shape), full(b3.shape),
                full(dec_W1.shape), full(db1.shape),
                full(dec_W2.shape), full(db2.shape),
                full(dec_W3.shape), full(db3.shape),
                full(codebooks.shape),
            ],
            out_specs=(
                pl.BlockSpec((bm, Din), lambda i: (i, 0)),
                pl.BlockSpec((G, K), lambda i: (0, 0)),
                scalar_spec, scalar_spec,
            ),
            out_shape=(
                jax.ShapeDtypeStruct((Bh, Din), jnp.float32),
                jax.ShapeDtypeStruct((G, K), jnp.float32),
                jax.ShapeDtypeStruct((1, 1), jnp.float32),
                jax.ShapeDtypeStruct((1, 1), jnp.float32),
            ),
            scratch_shapes=[
                pltpu.VMEM((G, K, D), jnp.bfloat16),
                pltpu.VMEM((G, K, D), jnp.bfloat16),
                pltpu.VMEM((G, K, D), jnp.bfloat16),
            ],
            compiler_params=pltpu.CompilerParams(
                dimension_semantics=("arbitrary",),
            ),
        )(xh, enc_W1, b1, enc_W2, b2, enc_W3, b3,
          dec_W1, db1, dec_W2, db2, dec_W3, db3, codebooks)

    p0, c0, l10, sq0 = _half(actions[:Bh].reshape(Bh, Din))
    p1, c1, l11, sq1 = _half(actions[Bh:].reshape(Bh, Din))

    def _finalize(c0_ref, c1_ref, l10_ref, l11_ref, sq0_ref, sq1_ref,
                  tot_ref, l1_ref, cbl_ref, pp_ref):
        counts = c0_ref[...] + c1_ref[...]
        probs = counts * (1.0 / Bsz)
        ppg = jnp.exp(-jnp.sum(probs * jnp.log(probs + 1e-10), axis=1,
                               keepdims=True))
        ppv = (ppg[0:1, :] + ppg[1:2, :]) * 0.5 if G == 2 else (
            jnp.sum(ppg).reshape(1, 1) / G)
        cblv = (sq0_ref[...] + sq1_ref[...]) * (1.0 / (Bsz * D)) * 1.25
        l1v = (l10_ref[...] + l11_ref[...]) * (1.0 / (Bsz * Din))
        l1_ref[...] = l1v
        cbl_ref[...] = cblv
        tot_ref[...] = l1v + cblv
        pp_ref[...] = ppv

    s11 = jax.ShapeDtypeStruct((1, 1), jnp.float32)
    tot, l1, cbl, pp = pl.pallas_call(
        _finalize,
        out_shape=(s11, s11, s11, s11),
    )(c0, c1, l10, l11, sq0, sq1)

    pred = jnp.concatenate([p0, p1], axis=0).reshape(Bsz, T, A)
    return (pred, tot[0, 0], l1[0, 0], cbl[0, 0], pp[0, 0])


# speedup vs baseline: 1.1591x; 1.1591x over previous
import jax
import jax.numpy as jnp
from jax.experimental import pallas as pl
from jax.experimental.pallas import tpu as pltpu

_BM = 1024


def _fused(x_ref, w1_ref, b1_ref, w2_ref, b2_ref, w3_ref, b3_ref,
           dw1_ref, db1_ref, dw2_ref, db2_ref, dw3_ref, db3_ref, cb_ref,
           pred_ref, tot_ref, l1_ref, cbl_ref, pp_ref,
           counts_ref, l1acc_ref, sqacc_ref,
           cbh_ref, cbm_ref, cbl_ref_in, cb2_ref, csq_ref):
    i = pl.program_id(0)
    nsteps = pl.num_programs(0)
    bm = x_ref.shape[0]
    G, K, D = cb_ref.shape
    Bsz = bm * nsteps
    Din = x_ref.shape[1]

    @pl.when(i == 0)
    def _init():
        counts_ref[...] = jnp.zeros_like(counts_ref)
        l1acc_ref[...] = jnp.zeros_like(l1acc_ref)
        sqacc_ref[...] = jnp.zeros_like(sqacc_ref)
        cb0 = cb_ref[...]
        hi = cb0.astype(jnp.bfloat16)
        r1 = cb0 - hi.astype(jnp.float32)
        mid = r1.astype(jnp.bfloat16)
        lo = r1 - mid.astype(jnp.float32)
        cbh_ref[...] = hi
        cbm_ref[...] = mid
        cbl_ref_in[...] = lo.astype(jnp.bfloat16)
        cb2_ref[...] = cb0 + cb0
        csq_ref[...] = jnp.sum(cb0 * cb0, axis=2)

    x = x_ref[...]
    h = jnp.maximum(
        jnp.dot(x, w1_ref[...], preferred_element_type=jnp.float32) + b1_ref[...], 0.0)
    h = jnp.maximum(
        jnp.dot(h, w2_ref[...], preferred_element_type=jnp.float32) + b2_ref[...], 0.0)
    z = jnp.dot(h, w3_ref[...], preferred_element_type=jnp.float32) + b3_ref[...]

    residual = z
    quantized = jnp.zeros_like(z)
    iota = jax.lax.broadcasted_iota(jnp.int32, (bm, K), 1)
    for g in range(G):
        rsq = jnp.sum(residual * residual, axis=1, keepdims=True)
        csq = csq_ref[g:g + 1, :]
        mm2 = jax.lax.dot_general(residual, cb2_ref[g], (((1,), (1,)), ((), ())),
                                  preferred_element_type=jnp.float32)
        dist = rsq - mm2 + csq
        m = jnp.min(dist, axis=1, keepdims=True)
        idxv = jnp.min(jnp.where(dist == m, iota, K), axis=1, keepdims=True)
        onehot = (iota == idxv).astype(jnp.bfloat16)
        if g < G - 1:
            q = (jnp.dot(onehot, cbh_ref[g], preferred_element_type=jnp.float32)
                 + jnp.dot(onehot, cbm_ref[g], preferred_element_type=jnp.float32)
                 + jnp.dot(onehot, cbl_ref_in[g], preferred_element_type=jnp.float32))
        else:
            q = (jnp.dot(onehot, cbh_ref[g], preferred_element_type=jnp.float32)
                 + jnp.dot(onehot, cbm_ref[g], preferred_element_type=jnp.float32))
        quantized = quantized + q
        residual = residual - q
        ones_row = jnp.ones((1, bm), jnp.bfloat16)
        counts_ref[g:g + 1, :] = counts_ref[g:g + 1, :] + jnp.dot(
            ones_row, onehot, preferred_element_type=jnp.float32)

    z_q = z + (quantized - z)
    hd = jnp.maximum(
        jnp.dot(z_q, dw1_ref[...], preferred_element_type=jnp.float32) + db1_ref[...], 0.0)
    hd = jnp.maximum(
        jnp.dot(hd, dw2_ref[...], preferred_element_type=jnp.float32) + db2_ref[...], 0.0)
    pred = jnp.dot(hd, dw3_ref[...], preferred_element_type=jnp.float32) + db3_ref[...]
    pred_ref[...] = pred

    diff = z - quantized
    sqacc_ref[...] = sqacc_ref[...] + jnp.sum(diff * diff).reshape(1, 1)
    l1acc_ref[...] = l1acc_ref[...] + jnp.sum(jnp.abs(pred - x)).reshape(1, 1)

    @pl.when(i == nsteps - 1)
    def _fin():
        probs = counts_ref[...] * (1.0 / Bsz)
        ppg = jnp.exp(-jnp.sum(probs * jnp.log(probs + 1e-10), axis=1,
                               keepdims=True))
        ppv = (ppg[0:1, :] + ppg[1:2, :]) * 0.5 if G == 2 else (
            jnp.sum(ppg).reshape(1, 1) / G)
        cbl = sqacc_ref[...] * (1.0 / (Bsz * D)) * 1.25
        l1v = l1acc_ref[...] * (1.0 / (Bsz * Din))
        l1_ref[...] = l1v
        cbl_ref[...] = cbl
        tot_ref[...] = l1v + cbl
        pp_ref[...] = ppv


def kernel(actions, enc_W1, enc_b1, enc_W2, enc_b2, enc_W3, enc_b3,
           dec_W1, dec_b1, dec_W2, dec_b2, dec_W3, dec_b3, codebooks):
    Bsz, T, A = actions.shape
    Din = T * A
    H = enc_W1.shape[1]
    G, K, D = codebooks.shape
    bm = _BM
    grid = Bsz // bm

    x = actions.reshape(Bsz, Din)
    b1 = enc_b1.reshape(1, H)
    b2 = enc_b2.reshape(1, H)
    b3 = enc_b3.reshape(1, D)
    db1 = dec_b1.reshape(1, H)
    db2 = dec_b2.reshape(1, H)
    db3 = dec_b3.reshape(1, Din)

    full = lambda shp: pl.BlockSpec(shp, lambda i: tuple(0 for _ in shp))
    scalar_spec = pl.BlockSpec((1, 1), lambda i: (0, 0))

    out = pl.pallas_call(
        _fused,
        grid=(grid,),
        in_specs=[
            pl.BlockSpec((bm, Din), lambda i: (i, 0)),
            full(enc_W1.shape), full(b1.shape),
            full(enc_W2.shape), full(b2.shape),
            full(enc_W3.shape), full(b3.shape),
            full(dec_W1.shape), full(db1.shape),
            full(dec_W2.shape), full(db2.shape),
            full(dec_W3.shape), full(db3.shape),
            full(codebooks.shape),
        ],
        out_specs=(
            pl.BlockSpec((bm, Din), lambda i: (i, 0)),
            scalar_spec, scalar_spec, scalar_spec, scalar_spec,
        ),
        out_shape=(
            jax.ShapeDtypeStruct((Bsz, Din), jnp.float32),
            jax.ShapeDtypeStruct((1, 1), jnp.float32),
            jax.ShapeDtypeStruct((1, 1), jnp.float32),
            jax.ShapeDtypeStruct((1, 1), jnp.float32),
            jax.ShapeDtypeStruct((1, 1), jnp.float32),
        ),
        scratch_shapes=[
            pltpu.VMEM((G, K), jnp.float32),
            pltpu.VMEM((1, 1), jnp.float32),
            pltpu.VMEM((1, 1), jnp.float32),
            pltpu.VMEM((G, K, D), jnp.bfloat16),
            pltpu.VMEM((G, K, D), jnp.bfloat16),
            pltpu.VMEM((G, K, D), jnp.bfloat16),
            pltpu.VMEM((G, K, D), jnp.float32),
            pltpu.VMEM((G, K), jnp.float32),
        ],
        compiler_params=pltpu.CompilerParams(
            dimension_semantics=("arbitrary",),
        ),
    )(x, enc_W1, b1, enc_W2, b2, enc_W3, b3,
      dec_W1, db1, dec_W2, db2, dec_W3, db3, codebooks)

    pred, tot, l1, cbl, pp = out
    return (pred.reshape(Bsz, T, A), tot[0, 0], l1[0, 0], cbl[0, 0], pp[0, 0])


# speedup vs baseline: 1.1674x; 1.0071x over previous
import jax
import jax.numpy as jnp
from jax.experimental import pallas as pl
from jax.experimental.pallas import tpu as pltpu

_BM = 1024


def _fused(x_ref, w1_ref, b1_ref, w2_ref, b2_ref, w3_ref, b3_ref,
           dw1_ref, db1_ref, dw2_ref, db2_ref, dw3_ref, db3_ref, cb_ref,
           pred_ref, tot_ref, l1_ref, cbl_ref, pp_ref,
           counts_ref, l1acc_ref, sqacc_ref,
           cbh_ref, cbm_ref, cbl_ref_in):
    i = pl.program_id(0)
    nsteps = pl.num_programs(0)
    bm = x_ref.shape[0]
    G, K, D = cb_ref.shape
    Bsz = bm * nsteps
    Din = x_ref.shape[1]

    @pl.when(i == 0)
    def _init():
        counts_ref[...] = jnp.zeros_like(counts_ref)
        l1acc_ref[...] = jnp.zeros_like(l1acc_ref)
        sqacc_ref[...] = jnp.zeros_like(sqacc_ref)
        cb0 = cb_ref[...]
        hi = cb0.astype(jnp.bfloat16)
        r1 = cb0 - hi.astype(jnp.float32)
        mid = r1.astype(jnp.bfloat16)
        lo = r1 - mid.astype(jnp.float32)
        cbh_ref[...] = hi
        cbm_ref[...] = mid
        cbl_ref_in[...] = lo.astype(jnp.bfloat16)

    x = x_ref[...]
    h = jnp.maximum(
        jnp.dot(x, w1_ref[...], preferred_element_type=jnp.float32) + b1_ref[...], 0.0)
    h = jnp.maximum(
        jnp.dot(h, w2_ref[...], preferred_element_type=jnp.float32) + b2_ref[...], 0.0)
    z = jnp.dot(h, w3_ref[...], preferred_element_type=jnp.float32) + b3_ref[...]

    residual = z
    quantized = jnp.zeros_like(z)
    iota = jax.lax.broadcasted_iota(jnp.int32, (bm, K), 1)
    for g in range(G):
        cb = cb_ref[g]
        rsq = jnp.sum(residual * residual, axis=1, keepdims=True)
        csq = jnp.sum(cb * cb, axis=1)[None, :]
        mm = jax.lax.dot_general(residual, cb, (((1,), (1,)), ((), ())),
                                 preferred_element_type=jnp.float32)
        dist = rsq - 2.0 * mm + csq
        m = jnp.min(dist, axis=1, keepdims=True)
        idxv = jnp.min(jnp.where(dist == m, iota, K), axis=1, keepdims=True)
        onehot = (iota == idxv).astype(jnp.bfloat16)
        if g < G - 1:
            q = (jnp.dot(onehot, cbh_ref[g], preferred_element_type=jnp.float32)
                 + jnp.dot(onehot, cbm_ref[g], preferred_element_type=jnp.float32)
                 + jnp.dot(onehot, cbl_ref_in[g], preferred_element_type=jnp.float32))
        else:
            q = (jnp.dot(onehot, cbh_ref[g], preferred_element_type=jnp.float32)
                 + jnp.dot(onehot, cbm_ref[g], preferred_element_type=jnp.float32))
        quantized = quantized + q
        residual = residual - q
        ones_row = jnp.ones((1, bm), jnp.bfloat16)
        counts_ref[g:g + 1, :] = counts_ref[g:g + 1, :] + jnp.dot(
            ones_row, onehot, preferred_element_type=jnp.float32)

    z_q = z + (quantized - z)
    hd = jnp.maximum(
        jnp.dot(z_q, dw1_ref[...], preferred_element_type=jnp.float32) + db1_ref[...], 0.0)
    hd = jnp.maximum(
        jnp.dot(hd, dw2_ref[...], preferred_element_type=jnp.float32) + db2_ref[...], 0.0)
    pred = jnp.dot(hd, dw3_ref[...], preferred_element_type=jnp.float32) + db3_ref[...]
    pred_ref[...] = pred

    diff = z - quantized
    sqacc_ref[...] = sqacc_ref[...] + jnp.sum(diff * diff).reshape(1, 1)
    l1acc_ref[...] = l1acc_ref[...] + jnp.sum(jnp.abs(pred - x)).reshape(1, 1)

    @pl.when(i == nsteps - 1)
    def _fin():
        probs = counts_ref[...] * (1.0 / Bsz)
        ppg = jnp.exp(-jnp.sum(probs * jnp.log(probs + 1e-10), axis=1,
                               keepdims=True))
        ppv = (ppg[0:1, :] + ppg[1:2, :]) * 0.5 if G == 2 else (
            jnp.sum(ppg).reshape(1, 1) / G)
        cbl = sqacc_ref[...] * (1.0 / (Bsz * D)) * 1.25
        l1v = l1acc_ref[...] * (1.0 / (Bsz * Din))
        l1_ref[...] = l1v
        cbl_ref[...] = cbl
        tot_ref[...] = l1v + cbl
        pp_ref[...] = ppv


def kernel(actions, enc_W1, enc_b1, enc_W2, enc_b2, enc_W3, enc_b3,
           dec_W1, dec_b1, dec_W2, dec_b2, dec_W3, dec_b3, codebooks):
    Bsz, T, A = actions.shape
    Din = T * A
    H = enc_W1.shape[1]
    G, K, D = codebooks.shape
    bm = _BM
    grid = Bsz // bm

    x = actions.reshape(Bsz, Din)
    b1 = enc_b1.reshape(1, H)
    b2 = enc_b2.reshape(1, H)
    b3 = enc_b3.reshape(1, D)
    db1 = dec_b1.reshape(1, H)
    db2 = dec_b2.reshape(1, H)
    db3 = dec_b3.reshape(1, Din)

    full = lambda shp: pl.BlockSpec(shp, lambda i: tuple(0 for _ in shp))
    scalar_spec = pl.BlockSpec((1, 1), lambda i: (0, 0))

    out = pl.pallas_call(
        _fused,
        grid=(grid,),
        in_specs=[
            pl.BlockSpec((bm, Din), lambda i: (i, 0)),
            full(enc_W1.shape), full(b1.shape),
            full(enc_W2.shape), full(b2.shape),
            full(enc_W3.shape), full(b3.shape),
            full(dec_W1.shape), full(db1.shape),
            full(dec_W2.shape), full(db2.shape),
            full(dec_W3.shape), full(db3.shape),
            full(codebooks.shape),
        ],
        out_specs=(
            pl.BlockSpec((bm, Din), lambda i: (i, 0)),
            scalar_spec, scalar_spec, scalar_spec, scalar_spec,
        ),
        out_shape=(
            jax.ShapeDtypeStruct((Bsz, Din), jnp.float32),
            jax.ShapeDtypeStruct((1, 1), jnp.float32),
            jax.ShapeDtypeStruct((1, 1), jnp.float32),
            jax.ShapeDtypeStruct((1, 1), jnp.float32),
            jax.ShapeDtypeStruct((1, 1), jnp.float32),
        ),
        scratch_shapes=[
            pltpu.VMEM((G, K), jnp.float32),
            pltpu.VMEM((1, 1), jnp.float32),
            pltpu.VMEM((1, 1), jnp.float32),
            pltpu.VMEM((G, K, D), jnp.bfloat16),
            pltpu.VMEM((G, K, D), jnp.bfloat16),
            pltpu.VMEM((G, K, D), jnp.bfloat16),
        ],
        compiler_params=pltpu.CompilerParams(
            dimension_semantics=("arbitrary",),
        ),
    )(x, enc_W1, b1, enc_W2, b2, enc_W3, b3,
      dec_W1, db1, dec_W2, db2, dec_W3, db3, codebooks)

    pred, tot, l1, cbl, pp = out
    return (pred.reshape(Bsz, T, A), tot[0, 0], l1[0, 0], cbl[0, 0], pp[0, 0])
